# Initial kernel scaffold; baseline (speedup 1.0000x reference)
import jax
import jax.numpy as jnp
from jax.experimental import pallas as pl


def kernel(hidden_states, gate_weight, e_score_correction_bias, w13, w2, shared_gate_up, shared_down):
    raise NotImplementedError("write your pallas kernel here")


# speedup vs baseline: 1.0791x; 1.0791x over previous
import jax
import jax.numpy as jnp
import numpy as np
from jax.experimental import pallas as pl
from jax.experimental.pallas import tpu as pltpu

T = 2048
D = 1024
E = 8
TOPK = 2
DFF = 512
NG = 4
TG = 2
NSH = 2
RSF = 2.5

TB = 256
NTB = T // TB

_NEG = np.float32(-1e30)


def _dot_nt(a, b, precision=None):
    return jax.lax.dot_general(
        a, b, (((1,), (1,)), ((), ())), preferred_element_type=jnp.float32,
        precision=precision,
    )


def _bf(a):
    return a.astype(jnp.bfloat16)


def _routing_combine(x_blk, gw, bias):
    nrows = x_blk.shape[0]
    logits = jax.lax.dot_general(
        x_blk, gw, (((1,), (1,)), ((), ())),
        preferred_element_type=jnp.float32,
        precision=jax.lax.Precision.HIGHEST,
    )
    scores = jax.nn.sigmoid(logits)
    sfc = scores + bias
    lane = jax.lax.broadcasted_iota(jnp.int32, (nrows, E), 1)
    glane = lane >> 1
    gvals = [
        jnp.sum(jnp.where(glane == g, sfc, 0.0), axis=1, keepdims=True)
        for g in range(NG)
    ]
    gsel = []
    for g in range(NG):
        rank = jnp.zeros_like(gvals[g])
        for g2 in range(NG):
            if g2 == g:
                continue
            beat = (gvals[g2] > gvals[g]) if g2 > g else (gvals[g2] >= gvals[g])
            rank += beat.astype(jnp.float32)
        gsel.append(rank <= (TG - 1))
    emask = jnp.zeros((nrows, E), dtype=jnp.bool_)
    for g in range(NG):
        emask = emask | ((glane == g) & gsel[g])
    ms = jnp.where(emask, sfc, _NEG)
    m1 = jnp.max(ms, axis=1, keepdims=True)
    i1 = jnp.min(jnp.where(ms == m1, lane, E), axis=1, keepdims=True)
    ms2 = jnp.where(lane == i1, _NEG, ms)
    m2 = jnp.max(ms2, axis=1, keepdims=True)
    i2 = jnp.min(jnp.where(ms2 == m2, lane, E), axis=1, keepdims=True)
    sel = (lane == i1) | (lane == i2)
    w = jnp.where(sel, scores, 0.0)
    return w / (jnp.sum(w, axis=1, keepdims=True) + 1e-20) * RSF


def _ffn_shared(x_blk, w13_blk, w2_blk):
    hp = jax.lax.Precision.HIGHEST
    gu = _dot_nt(x_blk, w13_blk, hp)
    half = w13_blk.shape[0] // 2
    g = gu[:, :half]
    u = gu[:, half:]
    act = g * jax.nn.sigmoid(g) * u
    return _dot_nt(act, w2_blk, hp)


def _ffn_routed(x_bf, w13_blk, w2_blk):
    gu = _dot_nt(x_bf, _bf(w13_blk))
    half = w13_blk.shape[0] // 2
    g = gu[:, :half]
    u = gu[:, half:]
    act = g * jax.nn.sigmoid(g) * u
    eo = _dot_nt(_bf(act), _bf(w2_blk))
    return _bf(eo).astype(jnp.float32)


def _routing_body(x_ref, gw_ref, bias_ref, comb_ref):
    comb_ref[...] = _routing_combine(x_ref[...], gw_ref[...], bias_ref[...])


def _body(x_ref, comb_in_ref, w13_ref, w2_ref, sgu_ref, sd_ref, out_ref):
    e = pl.program_id(1)
    x = x_ref[...]

    comb = comb_in_ref[...]
    lane = jax.lax.broadcasted_iota(jnp.int32, (TB, E), 1)
    col = jnp.sum(jnp.where(lane == e, comb, 0.0), axis=1, keepdims=True)
    colb = _bf(col).astype(jnp.float32)
    contrib = colb * _ffn_routed(_bf(x), w13_ref[0], w2_ref[0])

    @pl.when(e == 0)
    def _():
        out_ref[...] = _ffn_shared(x, sgu_ref[...], sd_ref[...]) + contrib

    @pl.when(e != 0)
    def _():
        out_ref[...] += contrib


def kernel(hidden_states, gate_weight, e_score_correction_bias, w13, w2,
           shared_gate_up, shared_down):
    bias2d = e_score_correction_bias.reshape(1, E)
    comb = pl.pallas_call(
        _routing_body,
        out_shape=jax.ShapeDtypeStruct((T, E), jnp.float32),
    )(hidden_states, gate_weight, bias2d)
    return pl.pallas_call(
        _body,
        grid=(NTB, E),
        in_specs=[
            pl.BlockSpec((TB, D), lambda tb, e: (tb, 0)),
            pl.BlockSpec((TB, E), lambda tb, e: (tb, 0)),
            pl.BlockSpec((1, 2 * DFF, D), lambda tb, e: (e, 0, 0)),
            pl.BlockSpec((1, D, DFF), lambda tb, e: (e, 0, 0)),
            pl.BlockSpec((2 * DFF * NSH, D), lambda tb, e: (0, 0)),
            pl.BlockSpec((D, DFF * NSH), lambda tb, e: (0, 0)),
        ],
        out_specs=pl.BlockSpec((TB, D), lambda tb, e: (tb, 0)),
        out_shape=jax.ShapeDtypeStruct((T, D), jnp.float32),
        compiler_params=pltpu.CompilerParams(
            dimension_semantics=("arbitrary", "arbitrary"),
        ),
    )(hidden_states, comb, w13, w2, shared_gate_up, shared_down)
